# Initial kernel scaffold; baseline (speedup 1.0000x reference)
import jax
import jax.numpy as jnp
from jax.experimental import pallas as pl


def kernel(x, rel_embedding):
    raise NotImplementedError("write your pallas kernel here")


# speedup vs baseline: 190.4883x; 190.4883x over previous
import jax
import jax.numpy as jnp
from jax.experimental import pallas as pl

MAX_LEN = 1024
DIM = 256
T = 1024
EPAD = 2048


def _rel_sum_body(e_ref, out_ref):
    i = jax.lax.broadcasted_iota(jnp.int32, (T, EPAD), 0)
    k = jax.lax.broadcasted_iota(jnp.int32, (T, EPAD), 1)
    s = i + k
    w = jnp.where((s >= MAX_LEN - 1) & (s <= 2 * MAX_LEN - 2), 1.0, 0.0)
    out_ref[:] = jnp.dot(w.astype(jnp.float32), e_ref[:],
                         preferred_element_type=jnp.float32)


def _add_body(rs_ref, x_ref, o_ref):
    o_ref[:] = x_ref[:] + rs_ref[:]


def kernel(x, rel_embedding):
    b, t, d = x.shape
    e_pad = jnp.concatenate(
        [rel_embedding, jnp.zeros((1, d), rel_embedding.dtype)], axis=0)

    rel_sum = pl.pallas_call(
        _rel_sum_body,
        out_shape=jax.ShapeDtypeStruct((t, d), jnp.float32),
    )(e_pad)

    out = pl.pallas_call(
        _add_body,
        grid=(b,),
        in_specs=[
            pl.BlockSpec((t, d), lambda i: (0, 0)),
            pl.BlockSpec((1, t, d), lambda i: (i, 0, 0)),
        ],
        out_specs=pl.BlockSpec((1, t, d), lambda i: (i, 0, 0)),
        out_shape=jax.ShapeDtypeStruct((b, t, d), x.dtype),
    )(rel_sum, x)
    return out


# speedup vs baseline: 257.5401x; 1.3520x over previous
import jax
import jax.numpy as jnp
from jax.experimental import pallas as pl
from jax.experimental.pallas import tpu as pltpu

MAX_LEN = 1024
DIM = 256
T = 1024
EPAD = 2048
BB = 2


def _fused_body(e_ref, x_ref, o_ref, rs_ref):
    @pl.when(pl.program_id(0) == 0)
    def _():
        i = jax.lax.broadcasted_iota(jnp.int32, (T, EPAD), 0)
        k = jax.lax.broadcasted_iota(jnp.int32, (T, EPAD), 1)
        s = i + k
        w = jnp.where((s >= MAX_LEN - 1) & (s <= 2 * MAX_LEN - 2), 1.0, 0.0)
        rs_ref[:] = jnp.dot(w.astype(jnp.float32), e_ref[:],
                            preferred_element_type=jnp.float32)

    o_ref[:] = x_ref[:] + rs_ref[:][None]


def kernel(x, rel_embedding):
    b, t, d = x.shape
    e_pad = jnp.concatenate(
        [rel_embedding, jnp.zeros((1, d), rel_embedding.dtype)], axis=0)

    return pl.pallas_call(
        _fused_body,
        grid=(b // BB,),
        in_specs=[
            pl.BlockSpec((EPAD, d), lambda i: (0, 0)),
            pl.BlockSpec((BB, t, d), lambda i: (i, 0, 0)),
        ],
        out_specs=pl.BlockSpec((BB, t, d), lambda i: (i, 0, 0)),
        out_shape=jax.ShapeDtypeStruct((b, t, d), x.dtype),
        scratch_shapes=[pltpu.VMEM((t, d), jnp.float32)],
    )(e_pad, x)


# speedup vs baseline: 273.0968x; 1.0604x over previous
import jax
import jax.numpy as jnp
from jax.experimental import pallas as pl
from jax.experimental.pallas import tpu as pltpu

MAX_LEN = 1024
DIM = 256
T = 1024
EPAD = 2048
BB = 4


def _fused_body(e_ref, x_ref, o_ref, rs_ref):
    @pl.when(pl.program_id(0) == 0)
    def _():
        i = jax.lax.broadcasted_iota(jnp.int32, (T, EPAD), 0)
        k = jax.lax.broadcasted_iota(jnp.int32, (T, EPAD), 1)
        s = i + k
        w = jnp.where((s >= MAX_LEN - 1) & (s <= 2 * MAX_LEN - 2), 1.0, 0.0)
        rs_ref[:] = jnp.dot(w.astype(jnp.float32), e_ref[:],
                            preferred_element_type=jnp.float32)

    o_ref[:] = x_ref[:] + rs_ref[:][None]


def kernel(x, rel_embedding):
    b, t, d = x.shape
    e_pad = jnp.concatenate(
        [rel_embedding, jnp.zeros((1, d), rel_embedding.dtype)], axis=0)

    return pl.pallas_call(
        _fused_body,
        grid=(b // BB,),
        in_specs=[
            pl.BlockSpec((EPAD, d), lambda i: (0, 0)),
            pl.BlockSpec((BB, t, d), lambda i: (i, 0, 0)),
        ],
        out_specs=pl.BlockSpec((BB, t, d), lambda i: (i, 0, 0)),
        out_shape=jax.ShapeDtypeStruct((b, t, d), x.dtype),
        scratch_shapes=[pltpu.VMEM((t, d), jnp.float32)],
    )(e_pad, x)


# speedup vs baseline: 306.6596x; 1.1229x over previous
import jax
import jax.numpy as jnp
from jax.experimental import pallas as pl
from jax.experimental.pallas import tpu as pltpu

MAX_LEN = 1024
DIM = 256
T = 1024
EPAD = 2048
BB = 8


def _fused_body(e_ref, x_ref, o_ref, rs_ref):
    @pl.when(pl.program_id(0) == 0)
    def _():
        i = jax.lax.broadcasted_iota(jnp.int32, (T, EPAD), 0)
        k = jax.lax.broadcasted_iota(jnp.int32, (T, EPAD), 1)
        s = i + k
        w = jnp.where((s >= MAX_LEN - 1) & (s <= 2 * MAX_LEN - 2), 1.0, 0.0)
        rs_ref[:] = jnp.dot(w.astype(jnp.float32), e_ref[:],
                            preferred_element_type=jnp.float32)

    o_ref[:] = x_ref[:] + rs_ref[:][None]


def kernel(x, rel_embedding):
    b, t, d = x.shape
    e_pad = jnp.concatenate(
        [rel_embedding, jnp.zeros((1, d), rel_embedding.dtype)], axis=0)

    return pl.pallas_call(
        _fused_body,
        grid=(b // BB,),
        in_specs=[
            pl.BlockSpec((EPAD, d), lambda i: (0, 0)),
            pl.BlockSpec((BB, t, d), lambda i: (i, 0, 0)),
        ],
        out_specs=pl.BlockSpec((BB, t, d), lambda i: (i, 0, 0)),
        out_shape=jax.ShapeDtypeStruct((b, t, d), x.dtype),
        scratch_shapes=[pltpu.VMEM((t, d), jnp.float32)],
    )(e_pad, x)
